# baseline (speedup 1.0000x reference)
import functools
import jax
import jax.numpy as jnp
from jax import lax
from jax.experimental import pallas as pl
from jax.experimental.pallas import tpu as pltpu

BR = 256


def _cdiv(a, b):
    return (a + b - 1) // b


def _pad_rows(x, m):
    r = x.shape[0]
    p = _cdiv(r, m) * m - r
    if p == 0:
        return x
    return jnp.pad(x, ((0, p),) + ((0, 0),) * (x.ndim - 1))


def _mm_body(x_ref, w_ref, b_ref, o_ref):
    o_ref[...] = jnp.dot(x_ref[...], w_ref[...],
                         preferred_element_type=jnp.float32) + b_ref[...]


def _mm(x, W, b):
    R, K = x.shape
    F = W.shape[1]
    return pl.pallas_call(
        _mm_body,
        grid=(R // BR,),
        in_specs=[
            pl.BlockSpec((BR, K), lambda i: (i, 0)),
            pl.BlockSpec((K, F), lambda i: (0, 0)),
            pl.BlockSpec((1, F), lambda i: (0, 0)),
        ],
        out_specs=pl.BlockSpec((BR, F), lambda i: (i, 0)),
        out_shape=jax.ShapeDtypeStruct((R, F), jnp.float32),
    )(x, W, b.reshape(1, F))


def _ln(v, g, b):
    mu = jnp.mean(v, axis=-1, keepdims=True)
    var = jnp.mean((v - mu) ** 2, axis=-1, keepdims=True)
    return (v - mu) / jnp.sqrt(var + 1e-5) * g + b


def _sigmoid(v):
    return 1.0 / (1.0 + jnp.exp(-v))


def _silu(v):
    return v * _sigmoid(v)


def _atom_body(x_ref, w_ref, b_ref, g_ref, bt_ref, o_ref):
    t = jnp.dot(x_ref[...], w_ref[...], preferred_element_type=jnp.float32)
    t = t + b_ref[...]
    o_ref[...] = _silu(_ln(t, g_ref[...], bt_ref[...]))


def _atom_mlp(x, W, b, g, bt):
    R, K = x.shape
    F = W.shape[1]
    return pl.pallas_call(
        _atom_body,
        grid=(R // BR,),
        in_specs=[
            pl.BlockSpec((BR, K), lambda i: (i, 0)),
            pl.BlockSpec((K, F), lambda i: (0, 0)),
            pl.BlockSpec((1, F), lambda i: (0, 0)),
            pl.BlockSpec((1, F), lambda i: (0, 0)),
            pl.BlockSpec((1, F), lambda i: (0, 0)),
        ],
        out_specs=pl.BlockSpec((BR, F), lambda i: (i, 0)),
        out_shape=jax.ShapeDtypeStruct((R, F), jnp.float32),
    )(x, W, b.reshape(1, F), g.reshape(1, F), bt.reshape(1, F))


def _rbf_body(r_ref, c_ref, W1_ref, b1_ref, g1_ref, bt1_ref,
              W2_ref, b2_ref, g2_ref, bt2_ref, o_ref, *, gamma, h1):
    r = r_ref[...]
    c = c_ref[...]
    rb = jnp.exp(-((r - c) * gamma) ** 2)
    t = jnp.dot(rb, W1_ref[...], preferred_element_type=jnp.float32) + b1_ref[...]
    tr = t[:, :h1]
    mu = jnp.mean(tr, axis=-1, keepdims=True)
    var = jnp.mean((tr - mu) ** 2, axis=-1, keepdims=True)
    tn = (tr - mu) / jnp.sqrt(var + 1e-5) * g1_ref[0, :h1] + bt1_ref[0, :h1]
    tn = _silu(tn)
    t2 = jnp.dot(tn, W2_ref[:h1, :], preferred_element_type=jnp.float32) + b2_ref[...]
    o_ref[...] = _silu(_ln(t2, g2_ref[...], bt2_ref[...]))


def _rbf_mlp(r, vmin, vmax, bins, W1, b1, g1, bt1, W2, b2, g2, bt2):
    R = r.shape[0]
    D = W2.shape[1]
    h1 = W1.shape[1]
    centers = jnp.linspace(vmin, vmax, bins)
    gamma = float(bins - 1) / (vmax - vmin)
    cpad = jnp.pad(centers, (0, 128 - bins), constant_values=1e30).reshape(1, 128)
    W1p = jnp.pad(W1, ((0, 128 - bins), (0, 128 - h1)))
    b1p = jnp.pad(b1, (0, 128 - h1)).reshape(1, 128)
    g1p = jnp.pad(g1, (0, 128 - h1)).reshape(1, 128)
    bt1p = jnp.pad(bt1, (0, 128 - h1)).reshape(1, 128)
    W2p = jnp.pad(W2, ((0, 128 - h1), (0, 0)))
    body = functools.partial(_rbf_body, gamma=gamma, h1=h1)
    return pl.pallas_call(
        body,
        grid=(R // BR,),
        in_specs=[
            pl.BlockSpec((BR, 1), lambda i: (i, 0)),
            pl.BlockSpec((1, 128), lambda i: (0, 0)),
            pl.BlockSpec((128, 128), lambda i: (0, 0)),
            pl.BlockSpec((1, 128), lambda i: (0, 0)),
            pl.BlockSpec((1, 128), lambda i: (0, 0)),
            pl.BlockSpec((1, 128), lambda i: (0, 0)),
            pl.BlockSpec((128, D), lambda i: (0, 0)),
            pl.BlockSpec((1, D), lambda i: (0, 0)),
            pl.BlockSpec((1, D), lambda i: (0, 0)),
            pl.BlockSpec((1, D), lambda i: (0, 0)),
        ],
        out_specs=pl.BlockSpec((BR, D), lambda i: (i, 0)),
        out_shape=jax.ShapeDtypeStruct((R, D), jnp.float32),
    )(r.reshape(R, 1), cpad, W1p, b1p, g1p, bt1p,
      W2p, b2.reshape(1, D), g2.reshape(1, D), bt2.reshape(1, D))


def _edge_stage_body(gs_ref, gd_ref, ge_ref, y_ref, g_ref, b_ref, bhs_ref,
                     yn_ref, vals_ref, sig_ref):
    m = gs_ref[...] + gd_ref[...] + ge_ref[...]
    sig = _sigmoid(m)
    yn_ref[...] = y_ref[...] + _silu(_ln(m, g_ref[...], b_ref[...]))
    sig_ref[...] = sig
    vals_ref[...] = sig * bhs_ref[...]


def _edge_stage(gs, gd, ge, y, bhs, g, b):
    R, D = gs.shape
    outs = pl.pallas_call(
        _edge_stage_body,
        grid=(R // BR,),
        in_specs=[pl.BlockSpec((BR, D), lambda i: (i, 0))] * 4 + [
            pl.BlockSpec((1, D), lambda i: (0, 0)),
            pl.BlockSpec((1, D), lambda i: (0, 0)),
        ] + [pl.BlockSpec((BR, D), lambda i: (i, 0))],
        out_specs=[pl.BlockSpec((BR, D), lambda i: (i, 0))] * 3,
        out_shape=[jax.ShapeDtypeStruct((R, D), jnp.float32)] * 3,
    )(gs, gd, ge, y, g.reshape(1, D), b.reshape(1, D), bhs)
    return outs


def _node_stage_body(x_ref, xw3_ref, num_ref, den_ref, g_ref, b_ref, o_ref):
    h = num_ref[...] / (den_ref[...] + 1e-6)
    o_ref[...] = x_ref[...] + _silu(_ln(xw3_ref[...] + h, g_ref[...], b_ref[...]))


def _node_stage(x, xw3, num, den, g, b):
    R, D = x.shape
    return pl.pallas_call(
        _node_stage_body,
        grid=(R // BR,),
        in_specs=[pl.BlockSpec((BR, D), lambda i: (i, 0))] * 4 + [
            pl.BlockSpec((1, D), lambda i: (0, 0)),
            pl.BlockSpec((1, D), lambda i: (0, 0)),
        ],
        out_specs=pl.BlockSpec((BR, D), lambda i: (i, 0)),
        out_shape=jax.ShapeDtypeStruct((R, D), jnp.float32),
    )(x, xw3, num, den, g.reshape(1, D), b.reshape(1, D))


def _mean_fc_body(x_ref, fcw_ref, fcb_ref, o_ref, acc_ref, *, nrows, nblocks):
    i = pl.program_id(0)

    @pl.when(i == 0)
    def _():
        acc_ref[...] = jnp.zeros_like(acc_ref)

    blk = x_ref[...]
    base = i * BR
    rid = base + lax.broadcasted_iota(jnp.int32, blk.shape, 0)
    blk = jnp.where(rid < nrows, blk, 0.0)
    acc_ref[...] += jnp.sum(blk, axis=0, keepdims=True)

    @pl.when(i == nblocks - 1)
    def _():
        h = acc_ref[...] / float(nrows)
        o_ref[...] = jnp.dot(h, fcw_ref[...],
                             preferred_element_type=jnp.float32) + fcb_ref[...]


def _mean_fc(x, nrows, fc_W, fc_b):
    R, D = x.shape
    nblocks = R // BR
    body = functools.partial(_mean_fc_body, nrows=nrows, nblocks=nblocks)
    out = pl.pallas_call(
        body,
        grid=(nblocks,),
        in_specs=[
            pl.BlockSpec((BR, D), lambda i: (i, 0)),
            pl.BlockSpec((D, 1), lambda i: (0, 0)),
            pl.BlockSpec((1, 1), lambda i: (0, 0)),
        ],
        out_specs=pl.BlockSpec((1, 1), lambda i: (0, 0)),
        out_shape=jax.ShapeDtypeStruct((1, 1), jnp.float32),
        scratch_shapes=[pltpu.VMEM((1, D), jnp.float32)],
    )(x, fc_W, fc_b.reshape(1, 1))
    return out.reshape(1)


def _gather_rows(table, idx):
    return table[idx]


def _segment_sum(vals, seg, nseg):
    return jax.ops.segment_sum(vals, seg, num_segments=nseg)


def _egc_layer(x, y, src, dst, W, b, ln, nseg, xrows):
    D = x.shape[1]
    Wcat = jnp.concatenate([W[0], W[1], W[4], W[3]], axis=1)
    bcat = jnp.concatenate([b[0], b[1], b[4], b[3]], axis=0)
    P = _mm(x, Wcat, bcat)
    ge = _mm(y, W[2], b[2])
    gs = _gather_rows(P[:, 0:D], src)
    gd = _gather_rows(P[:, D:2 * D], dst)
    bhs = _gather_rows(P[:, 2 * D:3 * D], src)
    yn, vals, sig = _edge_stage(gs, gd, ge, y, bhs, ln[2], ln[3])
    num = _segment_sum(vals, dst, nseg)
    den = _segment_sum(sig, dst, nseg)
    xn = _node_stage(x, P[:, 3 * D:4 * D], num, den, ln[0], ln[1])
    return xn, yn


def kernel(node_feats, bondlength, angle_cos, edge_index, lg_edge_index,
           atom_W, atom_b, atom_g, atom_bt,
           edge_W1, edge_b1, edge_g1, edge_bt1,
           edge_W2, edge_b2, edge_g2, edge_bt2,
           ang_W1, ang_b1, ang_g1, ang_bt1,
           ang_W2, ang_b2, ang_g2, ang_bt2,
           egc_W, egc_b, egc_ln, fc_W, fc_b):
    n = node_feats.shape[0]
    e = bondlength.shape[0]
    npad = _cdiv(n, BR) * BR
    src, dst = edge_index[0], edge_index[1]
    lsrc, ldst = lg_edge_index[0], lg_edge_index[1]

    nf = _pad_rows(node_feats, BR)
    nf = jnp.pad(nf, ((0, 0), (0, 128 - node_feats.shape[1])))
    aWp = jnp.pad(atom_W, ((0, 128 - atom_W.shape[0]), (0, 0)))
    x = _atom_mlp(nf, aWp, atom_b, atom_g, atom_bt)

    y = _rbf_mlp(bondlength, 0.0, 8.0, 80,
                 edge_W1, edge_b1, edge_g1, edge_bt1,
                 edge_W2, edge_b2, edge_g2, edge_bt2)
    z = _rbf_mlp(angle_cos, -1.0, 1.0, 40,
                 ang_W1, ang_b1, ang_g1, ang_bt1,
                 ang_W2, ang_b2, ang_g2, ang_bt2)

    for i in range(4):
        x, y = _egc_layer(x, y, src, dst, egc_W[2 * i], egc_b[2 * i],
                          egc_ln[2 * i], npad, n)
        y, z = _egc_layer(y, z, lsrc, ldst, egc_W[2 * i + 1], egc_b[2 * i + 1],
                          egc_ln[2 * i + 1], e, e)
    for i in range(8, 12):
        x, y = _egc_layer(x, y, src, dst, egc_W[i], egc_b[i], egc_ln[i], npad, n)

    return _mean_fc(x, n, fc_W, fc_b)


# speedup vs baseline: 1.2094x; 1.2094x over previous
import functools
import jax
import jax.numpy as jnp
from jax import lax
from jax.experimental import pallas as pl
from jax.experimental.pallas import tpu as pltpu

BR = 256


def _cdiv(a, b):
    return (a + b - 1) // b


def _pad_rows(x, m):
    r = x.shape[0]
    p = _cdiv(r, m) * m - r
    if p == 0:
        return x
    return jnp.pad(x, ((0, p),) + ((0, 0),) * (x.ndim - 1))


def _mm_body(x_ref, w_ref, b_ref, o_ref):
    o_ref[...] = jnp.dot(x_ref[...], w_ref[...],
                         preferred_element_type=jnp.float32) + b_ref[...]


def _mm(x, W, b):
    R, K = x.shape
    F = W.shape[1]
    return pl.pallas_call(
        _mm_body,
        grid=(R // BR,),
        in_specs=[
            pl.BlockSpec((BR, K), lambda i: (i, 0)),
            pl.BlockSpec((K, F), lambda i: (0, 0)),
            pl.BlockSpec((1, F), lambda i: (0, 0)),
        ],
        out_specs=pl.BlockSpec((BR, F), lambda i: (i, 0)),
        out_shape=jax.ShapeDtypeStruct((R, F), jnp.float32),
    )(x, W, b.reshape(1, F))


def _ln(v, g, b):
    mu = jnp.mean(v, axis=-1, keepdims=True)
    var = jnp.mean((v - mu) ** 2, axis=-1, keepdims=True)
    return (v - mu) / jnp.sqrt(var + 1e-5) * g + b


def _sigmoid(v):
    return 1.0 / (1.0 + jnp.exp(-v))


def _silu(v):
    return v * _sigmoid(v)


def _atom_body(x_ref, w_ref, b_ref, g_ref, bt_ref, o_ref):
    t = jnp.dot(x_ref[...], w_ref[...], preferred_element_type=jnp.float32)
    t = t + b_ref[...]
    o_ref[...] = _silu(_ln(t, g_ref[...], bt_ref[...]))


def _atom_mlp(x, W, b, g, bt):
    R, K = x.shape
    F = W.shape[1]
    return pl.pallas_call(
        _atom_body,
        grid=(R // BR,),
        in_specs=[
            pl.BlockSpec((BR, K), lambda i: (i, 0)),
            pl.BlockSpec((K, F), lambda i: (0, 0)),
            pl.BlockSpec((1, F), lambda i: (0, 0)),
            pl.BlockSpec((1, F), lambda i: (0, 0)),
            pl.BlockSpec((1, F), lambda i: (0, 0)),
        ],
        out_specs=pl.BlockSpec((BR, F), lambda i: (i, 0)),
        out_shape=jax.ShapeDtypeStruct((R, F), jnp.float32),
    )(x, W, b.reshape(1, F), g.reshape(1, F), bt.reshape(1, F))


def _rbf_body(r_ref, c_ref, W1_ref, b1_ref, g1_ref, bt1_ref,
              W2_ref, b2_ref, g2_ref, bt2_ref, o_ref, *, gamma, h1):
    r = r_ref[...]
    c = c_ref[...]
    rb = jnp.exp(-((r - c) * gamma) ** 2)
    t = jnp.dot(rb, W1_ref[...], preferred_element_type=jnp.float32) + b1_ref[...]
    tr = t[:, :h1]
    mu = jnp.mean(tr, axis=-1, keepdims=True)
    var = jnp.mean((tr - mu) ** 2, axis=-1, keepdims=True)
    tn = (tr - mu) / jnp.sqrt(var + 1e-5) * g1_ref[0, :h1] + bt1_ref[0, :h1]
    tn = _silu(tn)
    t2 = jnp.dot(tn, W2_ref[:h1, :], preferred_element_type=jnp.float32) + b2_ref[...]
    o_ref[...] = _silu(_ln(t2, g2_ref[...], bt2_ref[...]))


def _rbf_mlp(r, vmin, vmax, bins, W1, b1, g1, bt1, W2, b2, g2, bt2):
    R = r.shape[0]
    D = W2.shape[1]
    h1 = W1.shape[1]
    centers = jnp.linspace(vmin, vmax, bins)
    gamma = float(bins - 1) / (vmax - vmin)
    cpad = jnp.pad(centers, (0, 128 - bins), constant_values=1e30).reshape(1, 128)
    W1p = jnp.pad(W1, ((0, 128 - bins), (0, 128 - h1)))
    b1p = jnp.pad(b1, (0, 128 - h1)).reshape(1, 128)
    g1p = jnp.pad(g1, (0, 128 - h1)).reshape(1, 128)
    bt1p = jnp.pad(bt1, (0, 128 - h1)).reshape(1, 128)
    W2p = jnp.pad(W2, ((0, 128 - h1), (0, 0)))
    body = functools.partial(_rbf_body, gamma=gamma, h1=h1)
    return pl.pallas_call(
        body,
        grid=(R // BR,),
        in_specs=[
            pl.BlockSpec((BR, 1), lambda i: (i, 0)),
            pl.BlockSpec((1, 128), lambda i: (0, 0)),
            pl.BlockSpec((128, 128), lambda i: (0, 0)),
            pl.BlockSpec((1, 128), lambda i: (0, 0)),
            pl.BlockSpec((1, 128), lambda i: (0, 0)),
            pl.BlockSpec((1, 128), lambda i: (0, 0)),
            pl.BlockSpec((128, D), lambda i: (0, 0)),
            pl.BlockSpec((1, D), lambda i: (0, 0)),
            pl.BlockSpec((1, D), lambda i: (0, 0)),
            pl.BlockSpec((1, D), lambda i: (0, 0)),
        ],
        out_specs=pl.BlockSpec((BR, D), lambda i: (i, 0)),
        out_shape=jax.ShapeDtypeStruct((R, D), jnp.float32),
    )(r.reshape(R, 1), cpad, W1p, b1p, g1p, bt1p,
      W2p, b2.reshape(1, D), g2.reshape(1, D), bt2.reshape(1, D))


def _edge_stage_body(gs_ref, gd_ref, ge_ref, y_ref, g_ref, b_ref, bhs_ref,
                     yn_ref, vals_ref, sig_ref):
    m = gs_ref[...] + gd_ref[...] + ge_ref[...]
    sig = _sigmoid(m)
    yn_ref[...] = y_ref[...] + _silu(_ln(m, g_ref[...], b_ref[...]))
    sig_ref[...] = sig
    vals_ref[...] = sig * bhs_ref[...]


def _edge_stage(gs, gd, ge, y, bhs, g, b):
    R, D = gs.shape
    outs = pl.pallas_call(
        _edge_stage_body,
        grid=(R // BR,),
        in_specs=[pl.BlockSpec((BR, D), lambda i: (i, 0))] * 4 + [
            pl.BlockSpec((1, D), lambda i: (0, 0)),
            pl.BlockSpec((1, D), lambda i: (0, 0)),
        ] + [pl.BlockSpec((BR, D), lambda i: (i, 0))],
        out_specs=[pl.BlockSpec((BR, D), lambda i: (i, 0))] * 3,
        out_shape=[jax.ShapeDtypeStruct((R, D), jnp.float32)] * 3,
    )(gs, gd, ge, y, g.reshape(1, D), b.reshape(1, D), bhs)
    return outs


def _node_stage_body(x_ref, xw3_ref, num_ref, den_ref, g_ref, b_ref, o_ref):
    h = num_ref[...] / (den_ref[...] + 1e-6)
    o_ref[...] = x_ref[...] + _silu(_ln(xw3_ref[...] + h, g_ref[...], b_ref[...]))


def _node_stage(x, xw3, num, den, g, b):
    R, D = x.shape
    return pl.pallas_call(
        _node_stage_body,
        grid=(R // BR,),
        in_specs=[pl.BlockSpec((BR, D), lambda i: (i, 0))] * 4 + [
            pl.BlockSpec((1, D), lambda i: (0, 0)),
            pl.BlockSpec((1, D), lambda i: (0, 0)),
        ],
        out_specs=pl.BlockSpec((BR, D), lambda i: (i, 0)),
        out_shape=jax.ShapeDtypeStruct((R, D), jnp.float32),
    )(x, xw3, num, den, g.reshape(1, D), b.reshape(1, D))


def _mean_fc_body(x_ref, fcw_ref, fcb_ref, o_ref, acc_ref, *, nrows, nblocks):
    i = pl.program_id(0)

    @pl.when(i == 0)
    def _():
        acc_ref[...] = jnp.zeros_like(acc_ref)

    blk = x_ref[...]
    base = i * BR
    rid = base + lax.broadcasted_iota(jnp.int32, blk.shape, 0)
    blk = jnp.where(rid < nrows, blk, 0.0)
    acc_ref[...] += jnp.sum(blk, axis=0, keepdims=True)

    @pl.when(i == nblocks - 1)
    def _():
        h = acc_ref[...] / float(nrows)
        o_ref[...] = jnp.dot(h, fcw_ref[...],
                             preferred_element_type=jnp.float32) + fcb_ref[...]


def _mean_fc(x, nrows, fc_W, fc_b):
    R, D = x.shape
    nblocks = R // BR
    body = functools.partial(_mean_fc_body, nrows=nrows, nblocks=nblocks)
    out = pl.pallas_call(
        body,
        grid=(nblocks,),
        in_specs=[
            pl.BlockSpec((BR, D), lambda i: (i, 0)),
            pl.BlockSpec((D, 1), lambda i: (0, 0)),
            pl.BlockSpec((1, 1), lambda i: (0, 0)),
        ],
        out_specs=pl.BlockSpec((1, 1), lambda i: (0, 0)),
        out_shape=jax.ShapeDtypeStruct((1, 1), jnp.float32),
        scratch_shapes=[pltpu.VMEM((1, D), jnp.float32)],
    )(x, fc_W, fc_b.reshape(1, 1))
    return out.reshape(1)


from jax.experimental.pallas import tpu_sc as plsc

_NC, _NS = 2, 16
_NW = _NC * _NS


def _sc_gather(table, idx):
    R, D = table.shape
    E = idx.shape[0]
    CH = 200
    per_w = E // _NW
    n_ch = per_w // CH
    mesh = plsc.VectorSubcoreMesh(core_axis_name="c", subcore_axis_name="s")

    @functools.partial(
        pl.kernel, mesh=mesh,
        out_type=jax.ShapeDtypeStruct((E, D), jnp.float32),
        scratch_types=[
            pltpu.VMEM((CH,), jnp.int32),
            pltpu.VMEM((CH, D), jnp.float32),
            pltpu.SemaphoreType.DMA,
        ],
    )
    def k(table_hbm, idx_hbm, out_hbm, idx_v, rows_v, sem):
        wid = lax.axis_index("s") * _NC + lax.axis_index("c")
        base = wid * per_w

        def body(j, carry):
            b = base + j * CH
            pltpu.sync_copy(idx_hbm.at[pl.ds(b, CH)], idx_v)
            pltpu.async_copy(table_hbm.at[idx_v], rows_v, sem).wait()
            pltpu.sync_copy(rows_v, out_hbm.at[pl.ds(b, CH)])
            return carry

        lax.fori_loop(0, n_ch, body, 0)

    return k(table, idx)


def _sc_segsum_small(vals, seg, nseg):
    E, D = vals.shape
    Dh = D // _NC
    CH = 200
    per_t = E // _NS
    n_ch = per_t // CH
    stripe = nseg // _NS
    zeros = jnp.zeros((nseg, Dh), jnp.float32)
    mesh = plsc.VectorSubcoreMesh(core_axis_name="c", subcore_axis_name="s")

    @functools.partial(
        pl.kernel, mesh=mesh,
        out_type=jax.ShapeDtypeStruct((nseg, D), jnp.float32),
        scratch_types=[
            pltpu.VMEM_SHARED((nseg, Dh), jnp.float32),
            pltpu.VMEM((CH,), jnp.int32),
            pltpu.VMEM((CH, Dh), jnp.float32),
        ],
    )
    def k(vals_hbm, seg_hbm, zeros_hbm, out_hbm, accum, idx_v, rows_v):
        c = lax.axis_index("c")
        s = lax.axis_index("s")
        pltpu.sync_copy(zeros_hbm.at[pl.ds(s * stripe, stripe)],
                        accum.at[pl.ds(s * stripe, stripe)])
        plsc.subcore_barrier()

        def body(j, carry):
            e0 = s * per_t + j * CH
            pltpu.sync_copy(vals_hbm.at[pl.ds(e0, CH), pl.ds(c * Dh, Dh)],
                            rows_v)
            pltpu.sync_copy(seg_hbm.at[pl.ds(e0, CH)], idx_v)
            pltpu.sync_copy(rows_v, accum.at[idx_v], add=True)
            return carry

        lax.fori_loop(0, n_ch, body, 0)
        plsc.subcore_barrier()
        pltpu.sync_copy(accum.at[pl.ds(s * stripe, stripe)],
                        out_hbm.at[pl.ds(s * stripe, stripe),
                                   pl.ds(c * Dh, Dh)])

    return k(vals, seg, zeros)


def _gather_rows(table, idx):
    return _sc_gather(table, idx)


def _segment_sum(vals, seg, nseg):
    if nseg * (vals.shape[1] // _NC) * 4 <= 6 * 1024 * 1024:
        return _sc_segsum_small(vals, seg, nseg)
    return jax.ops.segment_sum(vals, seg, num_segments=nseg)


def _egc_layer(x, y, src, dst, W, b, ln, nseg, xrows):
    D = x.shape[1]
    Wcat = jnp.concatenate([W[0], W[1], W[4], W[3]], axis=1)
    bcat = jnp.concatenate([b[0], b[1], b[4], b[3]], axis=0)
    P = _mm(x, Wcat, bcat)
    ge = _mm(y, W[2], b[2])
    gs = _gather_rows(P[:, 0:D], src)
    gd = _gather_rows(P[:, D:2 * D], dst)
    bhs = _gather_rows(P[:, 2 * D:3 * D], src)
    yn, vals, sig = _edge_stage(gs, gd, ge, y, bhs, ln[2], ln[3])
    num = _segment_sum(vals, dst, nseg)
    den = _segment_sum(sig, dst, nseg)
    xn = _node_stage(x, P[:, 3 * D:4 * D], num, den, ln[0], ln[1])
    return xn, yn


def kernel(node_feats, bondlength, angle_cos, edge_index, lg_edge_index,
           atom_W, atom_b, atom_g, atom_bt,
           edge_W1, edge_b1, edge_g1, edge_bt1,
           edge_W2, edge_b2, edge_g2, edge_bt2,
           ang_W1, ang_b1, ang_g1, ang_bt1,
           ang_W2, ang_b2, ang_g2, ang_bt2,
           egc_W, egc_b, egc_ln, fc_W, fc_b):
    n = node_feats.shape[0]
    e = bondlength.shape[0]
    npad = _cdiv(n, BR) * BR
    src, dst = edge_index[0], edge_index[1]
    lsrc, ldst = lg_edge_index[0], lg_edge_index[1]

    nf = _pad_rows(node_feats, BR)
    nf = jnp.pad(nf, ((0, 0), (0, 128 - node_feats.shape[1])))
    aWp = jnp.pad(atom_W, ((0, 128 - atom_W.shape[0]), (0, 0)))
    x = _atom_mlp(nf, aWp, atom_b, atom_g, atom_bt)

    y = _rbf_mlp(bondlength, 0.0, 8.0, 80,
                 edge_W1, edge_b1, edge_g1, edge_bt1,
                 edge_W2, edge_b2, edge_g2, edge_bt2)
    z = _rbf_mlp(angle_cos, -1.0, 1.0, 40,
                 ang_W1, ang_b1, ang_g1, ang_bt1,
                 ang_W2, ang_b2, ang_g2, ang_bt2)

    for i in range(4):
        x, y = _egc_layer(x, y, src, dst, egc_W[2 * i], egc_b[2 * i],
                          egc_ln[2 * i], npad, n)
        y, z = _egc_layer(y, z, lsrc, ldst, egc_W[2 * i + 1], egc_b[2 * i + 1],
                          egc_ln[2 * i + 1], e, e)
    for i in range(8, 12):
        x, y = _egc_layer(x, y, src, dst, egc_W[i], egc_b[i], egc_ln[i], npad, n)

    return _mean_fc(x, n, fc_W, fc_b)
